# baseline (speedup 1.0000x reference)
import jax
import jax.numpy as jnp
from jax import lax
from jax.experimental import pallas as pl
from jax.experimental.pallas import tpu as pltpu
from jax.experimental.pallas import tpu_sc as plsc

B = 16384
HID = 1024
HOP = 8
K = 4
NREL_PAD = 1024
LAMBDA_NOISE = 1.0

_HHALF = HID // 2


def _rel_table_body(r_ref, w_ref, b_ref, vi_ref):
    v = (
        jnp.dot(r_ref[...], w_ref[...], preferred_element_type=jnp.float32)
        + b_ref[...]
    )
    lo = v[:, :_HHALF].astype(jnp.bfloat16).astype(jnp.float32)
    hi = v[:, _HHALF:].astype(jnp.bfloat16).astype(jnp.float32)
    lo_bits = jax.lax.bitcast_convert_type(lo, jnp.uint32) >> 16
    hi_bits = jax.lax.bitcast_convert_type(hi, jnp.uint32) & jnp.uint32(
        0xFFFF0000
    )
    vi_ref[...] = jax.lax.bitcast_convert_type(lo_bits | hi_bits, jnp.int32)


def _rel_table(r_bf, w1b_bf, b1_row):
    return pl.pallas_call(
        _rel_table_body,
        out_shape=jax.ShapeDtypeStruct((NREL_PAD, _HHALF), jnp.int32),
    )(r_bf, w1b_bf, b1_row)


_NC = 2
_NS = 16
_NW = _NC * _NS
_NSPLIT = 2
_BS = B // _NSPLIT
_BPW = _BS // _NW
_CH = 32
_NCHUNK = _BPW // _CH
_TOT = 2 * _NCHUNK


def _sc_gather_body(table_hbm, ids_hbm, out_hbm, idx, buf0, buf1, sem0, sem1):
    wid = lax.axis_index("s") * _NC + lax.axis_index("c")
    base = wid * _BPW
    pltpu.sync_copy(ids_hbm.at[pl.ds(base, _BPW)], idx)
    bufs = (buf0, buf1)
    sems = (sem0, sem1)

    cps = [None] * _NCHUNK
    cps[0] = pltpu.async_copy(
        table_hbm.at[idx.at[pl.ds(0, _CH)]], bufs[0], sems[0]
    )
    for k in range(_NCHUNK):
        cur = k & 1
        if k + 1 < _NCHUNK:
            cps[k + 1] = pltpu.async_copy(
                table_hbm.at[idx.at[pl.ds((k + 1) * _CH, _CH)]],
                bufs[1 - cur], sems[1 - cur],
            )
        cps[k].wait()
        pltpu.sync_copy(bufs[cur], out_hbm.at[pl.ds(base + k * _CH, _CH)])


def _sc_gather(table, ids_half):
    width = table.shape[1]
    f = pl.kernel(
        _sc_gather_body,
        out_type=jax.ShapeDtypeStruct((_BS, width), table.dtype),
        mesh=plsc.VectorSubcoreMesh(core_axis_name="c", subcore_axis_name="s"),
        scratch_types=[
            pltpu.VMEM((_BPW,), jnp.int32),
            pltpu.VMEM((_CH, width), table.dtype),
            pltpu.VMEM((_CH, width), table.dtype),
            pltpu.SemaphoreType.DMA,
            pltpu.SemaphoreType.DMA,
        ],
    )
    return f(table, ids_half)


_TILE = 512
_NTILE = _BS // _TILE


def _mlp_body(ge_ref, gv_ref, w_ref, out_ref, acc_ref):
    i = pl.program_id(0)

    @pl.when(i == 0)
    def _():
        acc_ref[...] = jnp.zeros_like(acc_ref)

    h = jnp.dot(
        ge_ref[...].astype(jnp.bfloat16), w_ref[...],
        preferred_element_type=jnp.float32,
    )
    gvi = gv_ref[...]
    lo_f = jax.lax.bitcast_convert_type(gvi << 16, jnp.float32)
    hi_f = jax.lax.bitcast_convert_type(gvi & jnp.int32(-65536), jnp.float32)
    gv = jnp.concatenate([lo_f, hi_f], axis=1)
    h = jnp.maximum(h + gv, 0.0)
    acc_ref[...] += jnp.sum(h, axis=0, keepdims=True)

    @pl.when(i == pl.num_programs(0) - 1)
    def _():
        out_ref[...] = acc_ref[...]


def _mlp_sum(ge, gv, w1a_bf):
    return pl.pallas_call(
        _mlp_body,
        grid=(_NTILE,),
        in_specs=[
            pl.BlockSpec((_TILE, HID), lambda i: (i, 0)),
            pl.BlockSpec((_TILE, _HHALF), lambda i: (i, 0)),
            pl.BlockSpec((HID, HID), lambda i: (0, 0)),
        ],
        out_specs=pl.BlockSpec((1, HID), lambda i: (0, 0)),
        out_shape=jax.ShapeDtypeStruct((1, HID), jnp.float32),
        scratch_shapes=[pltpu.VMEM((1, HID), jnp.float32)],
    )(ge, gv, w1a_bf)


def _route_body(hs_ref, w2_ref, b2_ref, hop_ref, wn_ref, eps_ref, g_ref, q_ref):
    hsum = jnp.sum(hs_ref[...], axis=0, keepdims=True)
    c = (
        jnp.dot(hsum * (1.0 / B), w2_ref[...],
                preferred_element_type=jnp.float32)
        + b2_ref[...]
    )
    qc = jnp.sum(hop_ref[...] * c, axis=1, keepdims=True)
    s = jnp.sum(c * wn_ref[...], axis=1, keepdims=True)
    sigma = jnp.maximum(s, 0.0) + jnp.log1p(jnp.exp(-jnp.abs(s)))
    q = qc + LAMBDA_NOISE * eps_ref[...] * sigma

    iot = lax.broadcasted_iota(jnp.int32, (HOP, 1), 0)
    qw = q
    vals, idxs = [], []
    for _ in range(K):
        m = jnp.max(qw, axis=0, keepdims=True)
        ii = jnp.min(jnp.where(qw == m, iot, HOP), axis=0, keepdims=True)
        vals.append(m)
        idxs.append(ii)
        qw = jnp.where(iot == ii, -jnp.inf, qw)
    es = [jnp.exp(v - vals[0]) for v in vals]
    z = es[0] + es[1] + es[2] + es[3]
    g = jnp.zeros((HOP, 1), jnp.float32)
    for ii, e in zip(idxs, es):
        g = g + jnp.where(iot == ii, e / z, 0.0)
    g_ref[...] = g
    q_ref[...] = q


def _route(hs_stack, W2, b2_row, hop_embed, wn_row, eps_col):
    return pl.pallas_call(
        _route_body,
        out_shape=(
            jax.ShapeDtypeStruct((HOP, 1), jnp.float32),
            jax.ShapeDtypeStruct((HOP, 1), jnp.float32),
        ),
    )(hs_stack, W2, b2_row, hop_embed, wn_row, eps_col)


def kernel(subs, rels, entity_embed, relation_embed, hop_embed,
           W1, b1, W2, b2, w_n, noise_eps):
    w1a_bf = W1[:HID].astype(jnp.bfloat16)
    w1b_bf = W1[HID:].astype(jnp.bfloat16)
    r_bf = jnp.pad(
        relation_embed, ((0, NREL_PAD - relation_embed.shape[0]), (0, 0))
    ).astype(jnp.bfloat16)

    ges = [
        _sc_gather(entity_embed,
                   lax.slice_in_dim(subs, s * _BS, (s + 1) * _BS))
        for s in range(_NSPLIT)
    ]
    v = _rel_table(r_bf, w1b_bf, b1.reshape(1, HID))
    gvs = [
        _sc_gather(v, lax.slice_in_dim(rels, s * _BS, (s + 1) * _BS))
        for s in range(_NSPLIT)
    ]
    hsums = [_mlp_sum(ges[s], gvs[s], w1a_bf) for s in range(_NSPLIT)]
    g, q = _route(
        jnp.concatenate(hsums, axis=0), W2, b2.reshape(1, HID), hop_embed,
        w_n.reshape(1, HID), noise_eps.reshape(HOP, 1),
    )
    return (g.reshape(HOP), q.reshape(HOP))


# speedup vs baseline: 1.0249x; 1.0249x over previous
import jax
import jax.numpy as jnp
from jax import lax
from jax.experimental import pallas as pl
from jax.experimental.pallas import tpu as pltpu
from jax.experimental.pallas import tpu_sc as plsc

B = 16384
HID = 1024
HOP = 8
K = 4
NREL_PAD = 1024
LAMBDA_NOISE = 1.0

_HHALF = HID // 2


def _rel_table_body(r_ref, w_ref, b_ref, vi_ref):
    v = (
        jnp.dot(r_ref[...], w_ref[...], preferred_element_type=jnp.float32)
        + b_ref[...]
    )
    lo = v[:, :_HHALF].astype(jnp.bfloat16).astype(jnp.float32)
    hi = v[:, _HHALF:].astype(jnp.bfloat16).astype(jnp.float32)
    lo_bits = jax.lax.bitcast_convert_type(lo, jnp.uint32) >> 16
    hi_bits = jax.lax.bitcast_convert_type(hi, jnp.uint32) & jnp.uint32(
        0xFFFF0000
    )
    vi_ref[...] = jax.lax.bitcast_convert_type(lo_bits | hi_bits, jnp.int32)


def _rel_table(r_bf, w1b_bf, b1_row):
    return pl.pallas_call(
        _rel_table_body,
        out_shape=jax.ShapeDtypeStruct((NREL_PAD, _HHALF), jnp.int32),
    )(r_bf, w1b_bf, b1_row)


_NC = 2
_NS = 16
_NW = _NC * _NS
_NSPLIT = 2
_BS = B // _NSPLIT
_BPW = _BS // _NW
_CH = 32
_NCHUNK = _BPW // _CH
_TOT = 2 * _NCHUNK


def _sc_gather_body(table_hbm, ids_hbm, out_hbm, idx, buf0, buf1, sem0, sem1):
    wid = lax.axis_index("s") * _NC + lax.axis_index("c")
    base = wid * _BPW
    pltpu.sync_copy(ids_hbm.at[pl.ds(base, _BPW)], idx)
    bufs = (buf0, buf1)
    sems = (sem0, sem1)

    cps = [None] * _NCHUNK
    cps[0] = pltpu.async_copy(
        table_hbm.at[idx.at[pl.ds(0, _CH)]], bufs[0], sems[0]
    )
    for k in range(_NCHUNK):
        cur = k & 1
        if k + 1 < _NCHUNK:
            cps[k + 1] = pltpu.async_copy(
                table_hbm.at[idx.at[pl.ds((k + 1) * _CH, _CH)]],
                bufs[1 - cur], sems[1 - cur],
            )
        cps[k].wait()
        pltpu.sync_copy(bufs[cur], out_hbm.at[pl.ds(base + k * _CH, _CH)])


def _sc_gather(table, ids_half):
    width = table.shape[1]
    f = pl.kernel(
        _sc_gather_body,
        out_type=jax.ShapeDtypeStruct((_BS, width), table.dtype),
        mesh=plsc.VectorSubcoreMesh(core_axis_name="c", subcore_axis_name="s"),
        scratch_types=[
            pltpu.VMEM((_BPW,), jnp.int32),
            pltpu.VMEM((_CH, width), table.dtype),
            pltpu.VMEM((_CH, width), table.dtype),
            pltpu.SemaphoreType.DMA,
            pltpu.SemaphoreType.DMA,
        ],
    )
    return f(table, ids_half)


_TILE = 512
_NTILE = _BS // _TILE


def _mlp_body(ge_ref, gv_ref, w_ref, out_ref, acc_ref):
    i = pl.program_id(0)

    @pl.when(i == 0)
    def _():
        acc_ref[...] = jnp.zeros_like(acc_ref)

    h = jnp.dot(
        ge_ref[...].astype(jnp.bfloat16), w_ref[...],
        preferred_element_type=jnp.float32,
    )
    gvi = gv_ref[...]
    lo_f = jax.lax.bitcast_convert_type(gvi << 16, jnp.float32)
    hi_f = jax.lax.bitcast_convert_type(gvi & jnp.int32(-65536), jnp.float32)
    gv = jnp.concatenate([lo_f, hi_f], axis=1)
    h = jnp.maximum(h + gv, 0.0)
    acc_ref[...] += jnp.sum(h, axis=0, keepdims=True)

    @pl.when(i == pl.num_programs(0) - 1)
    def _():
        out_ref[...] = acc_ref[...]


def _mlp_sum(ge, gv, w1a_bf):
    return pl.pallas_call(
        _mlp_body,
        grid=(_NTILE,),
        in_specs=[
            pl.BlockSpec((_TILE, HID), lambda i: (i, 0)),
            pl.BlockSpec((_TILE, _HHALF), lambda i: (i, 0)),
            pl.BlockSpec((HID, HID), lambda i: (0, 0)),
        ],
        out_specs=pl.BlockSpec((1, HID), lambda i: (0, 0)),
        out_shape=jax.ShapeDtypeStruct((1, HID), jnp.float32),
        scratch_shapes=[pltpu.VMEM((1, HID), jnp.float32)],
    )(ge, gv, w1a_bf)


def _route_body(hs_ref, w2_ref, b2_ref, hop_ref, wn_ref, eps_ref, g_ref, q_ref):
    hsum = jnp.sum(hs_ref[...], axis=0, keepdims=True)
    c = (
        jnp.dot(hsum * (1.0 / B), w2_ref[...],
                preferred_element_type=jnp.float32)
        + b2_ref[...]
    )
    qc = jnp.sum(hop_ref[...] * c, axis=1, keepdims=True)
    s = jnp.sum(c * wn_ref[...], axis=1, keepdims=True)
    sigma = jnp.maximum(s, 0.0) + jnp.log1p(jnp.exp(-jnp.abs(s)))
    q = qc + LAMBDA_NOISE * eps_ref[...] * sigma

    iot = lax.broadcasted_iota(jnp.int32, (HOP, 1), 0)
    qw = q
    vals, idxs = [], []
    for _ in range(K):
        m = jnp.max(qw, axis=0, keepdims=True)
        ii = jnp.min(jnp.where(qw == m, iot, HOP), axis=0, keepdims=True)
        vals.append(m)
        idxs.append(ii)
        qw = jnp.where(iot == ii, -jnp.inf, qw)
    es = [jnp.exp(v - vals[0]) for v in vals]
    z = es[0] + es[1] + es[2] + es[3]
    g = jnp.zeros((HOP, 1), jnp.float32)
    for ii, e in zip(idxs, es):
        g = g + jnp.where(iot == ii, e / z, 0.0)
    g_ref[...] = g
    q_ref[...] = q


def _route(hs_stack, W2, b2_row, hop_embed, wn_row, eps_col):
    return pl.pallas_call(
        _route_body,
        out_shape=(
            jax.ShapeDtypeStruct((HOP, 1), jnp.float32),
            jax.ShapeDtypeStruct((HOP, 1), jnp.float32),
        ),
    )(hs_stack, W2, b2_row, hop_embed, wn_row, eps_col)


def kernel(subs, rels, entity_embed, relation_embed, hop_embed,
           W1, b1, W2, b2, w_n, noise_eps):
    w1a_bf = W1[:HID].astype(jnp.bfloat16)
    w1b_bf = W1[HID:].astype(jnp.bfloat16)
    r_bf = jnp.pad(
        relation_embed, ((0, NREL_PAD - relation_embed.shape[0]), (0, 0))
    ).astype(jnp.bfloat16)

    ges = [
        _sc_gather(entity_embed,
                   lax.slice_in_dim(subs, s * _BS, (s + 1) * _BS))
        for s in range(_NSPLIT)
    ]
    r_bf, w1b_bf, b1_row, _ = lax.optimization_barrier(
        (r_bf, w1b_bf, b1.reshape(1, HID), ges[0])
    )
    v = _rel_table(r_bf, w1b_bf, b1_row)
    gvs = [
        _sc_gather(v, lax.slice_in_dim(rels, s * _BS, (s + 1) * _BS))
        for s in range(_NSPLIT)
    ]
    hsums = [_mlp_sum(ges[s], gvs[s], w1a_bf) for s in range(_NSPLIT)]
    g, q = _route(
        jnp.concatenate(hsums, axis=0), W2, b2.reshape(1, HID), hop_embed,
        w_n.reshape(1, HID), noise_eps.reshape(HOP, 1),
    )
    return (g.reshape(HOP), q.reshape(HOP))
